# baseline (speedup 1.0000x reference)
import functools
import jax
import jax.numpy as jnp
from jax import lax
from jax.experimental import pallas as pl
from jax.experimental.pallas import tpu as pltpu
from jax.experimental.pallas import tpu_sc as plsc

N = 10000
D = 128
DH = 64
E = 320000
NSUB = 16
CHUNK = 128
CHUNKS = 158
QUADS = (CHUNKS - 6) // 4
E_SUB = CHUNKS * CHUNK
E_PAD = E_SUB * NSUB
N_PAD = 10112
RPW = N_PAD // NSUB
BM = 2000


def _make_sc_agg(with_deg):
    mesh = plsc.VectorSubcoreMesh(core_axis_name="c", subcore_axis_name="s")
    out_type = [jax.ShapeDtypeStruct((N_PAD, DH), jnp.float32)
                for _ in range(4)]
    scratch = [
        pltpu.VMEM((CHUNKS, CHUNK), jnp.int32),
        pltpu.VMEM((CHUNKS, CHUNK), jnp.int32),
    ] + [pltpu.VMEM((CHUNK, DH), jnp.float32)] * 4 + [
        pltpu.VMEM_SHARED((N_PAD, DH), jnp.float32),
    ] + [pltpu.SemaphoreType.DMA] * 8
    if with_deg:
        out_type.append(jax.ShapeDtypeStruct((N_PAD, 16), jnp.float32))
        scratch += [
            pltpu.VMEM((CHUNK, 16), jnp.float32),
            pltpu.VMEM_SHARED((N_PAD, 16), jnp.float32),
        ]

    @functools.partial(
        pl.kernel, mesh=mesh, out_type=out_type, scratch_types=scratch,
        compiler_params=pltpu.CompilerParams(use_tc_tiling_on_sc=False))
    def sc_agg(*refs):
        if with_deg:
            (hl_a, hl_b, hh_a, hh_b, col3, row3, zeros_hbm, zeros16_hbm,
             ones_hbm, aggl_a, aggl_b, aggh_a, aggh_b, deg_out,
             col_v, row_v, b0, b1, b2, b3, acc,
             g0, g1, g2, g3, s0, s1, s2, s3, ones_v, accd) = refs
        else:
            (hl_a, hl_b, hh_a, hh_b, col3, row3, zeros_hbm,
             aggl_a, aggl_b, aggh_a, aggh_b,
             col_v, row_v, b0, b1, b2, b3, acc,
             g0, g1, g2, g3, s0, s1, s2, s3) = refs
        bufs = (b0, b1, b2, b3)
        gsems = (g0, g1, g2, g3)
        ssems = (s0, s1, s2, s3)

        cid = lax.axis_index("c")
        sid = lax.axis_index("s")
        sl = pl.ds(sid * RPW, RPW)

        pltpu.sync_copy(col3.at[sid], col_v)
        pltpu.sync_copy(row3.at[sid], row_v)
        pltpu.sync_copy(zeros_hbm.at[sl], acc.at[sl])
        if with_deg:
            pltpu.sync_copy(zeros16_hbm.at[sl], accd.at[sl])
            pltpu.sync_copy(ones_hbm, ones_v)
        plsc.subcore_barrier()

        def run(src):
            dummy = src.at[pl.ds(0, CHUNK)]

            def gfire(j, x):
                pltpu.async_copy(src.at[col_v.at[j]], bufs[x], gsems[x])

            def gwait(x):
                pltpu.make_async_copy(dummy, bufs[x], gsems[x]).wait()

            def sfire(j, x):
                pltpu.async_copy(bufs[x], acc.at[row_v.at[j]], ssems[x],
                                 add=True)

            def swait(x):
                pltpu.make_async_copy(bufs[x], acc.at[pl.ds(0, CHUNK)],
                                      ssems[x]).wait()

            gfire(0, 0)
            gfire(1, 1)
            gwait(0); sfire(0, 0); gfire(2, 2)
            gwait(1); sfire(1, 1); gfire(3, 3)

            def step(t, carry):
                j = 4 * t + 2
                for d in range(4):
                    x = (2 + d) % 4
                    gwait(x)
                    sfire(j + d, x)
                    swait((x + 2) % 4)
                    gfire(j + d + 2, (x + 2) % 4)
                return carry

            lax.fori_loop(0, QUADS, step, 0)
            gwait(2); sfire(CHUNKS - 4, 2); swait(0); gfire(CHUNKS - 2, 0)
            gwait(3); sfire(CHUNKS - 3, 3); swait(1); gfire(CHUNKS - 1, 1)
            gwait(0); sfire(CHUNKS - 2, 0); swait(2)
            gwait(1); sfire(CHUNKS - 1, 1); swait(3)
            swait(0); swait(1)

        for p, (srcs, outs) in enumerate((((hl_a, hh_a), (aggl_a, aggh_a)),
                                          ((hl_b, hh_b), (aggl_b, aggh_b)))):
            @pl.when(cid == 0)
            def _(srcs=srcs):
                run(srcs[0])

            @pl.when(cid == 1)
            def _(srcs=srcs):
                run(srcs[1])

            if with_deg and p == 0:
                @pl.when(cid == 0)
                def _():
                    def dstep(j, carry):
                        pltpu.sync_copy(ones_v, accd.at[row_v.at[j]],
                                        add=True)
                        return carry
                    lax.fori_loop(0, CHUNKS, dstep, 0)

            plsc.subcore_barrier()

            @pl.when(cid == 0)
            def _(outs=outs):
                pltpu.sync_copy(acc.at[sl], outs[0].at[sl])

            @pl.when(cid == 1)
            def _(outs=outs):
                pltpu.sync_copy(acc.at[sl], outs[1].at[sl])

            if with_deg and p == 0:
                @pl.when(cid == 0)
                def _():
                    pltpu.sync_copy(accd.at[sl], deg_out.at[sl])

            if p == 0:
                pltpu.sync_copy(zeros_hbm.at[sl], acc.at[sl])
                plsc.subcore_barrier()

    return sc_agg


_sc_agg_deg = _make_sc_agg(True)
_sc_agg = _make_sc_agg(False)


def _pre_body(x_ref, wl_ref, wh_ref, wm_ref,
              la_ref, lb_ref, ha_ref, hb_ref, om_ref):
    h = x_ref[...]
    rl = jnp.dot(h, wl_ref[...], preferred_element_type=jnp.float32)
    rh = jnp.dot(h, wh_ref[...], preferred_element_type=jnp.float32)
    la_ref[...] = rl[:, :DH]
    lb_ref[...] = rl[:, DH:]
    ha_ref[...] = rh[:, :DH]
    hb_ref[...] = rh[:, DH:]
    om_ref[...] = jnp.dot(h, wm_ref[...], preferred_element_type=jnp.float32)


def _pre(x, wl, wh, wm):
    grid = (N // BM,)
    row_spec = pl.BlockSpec((BM, D), lambda i: (i, 0))
    half_spec = pl.BlockSpec((BM, DH), lambda i: (i, 0))
    w_spec = pl.BlockSpec((D, D), lambda i: (0, 0))
    return pl.pallas_call(
        _pre_body,
        grid=grid,
        in_specs=[row_spec, w_spec, w_spec, w_spec],
        out_specs=[half_spec, half_spec, half_spec, half_spec, row_spec],
        out_shape=[jax.ShapeDtypeStruct((N, DH), jnp.float32)] * 4
        + [jax.ShapeDtypeStruct((N, D), jnp.float32)],
    )(x, wl, wh, wm)


def _post_body(hla_ref, hlb_ref, hha_ref, hhb_ref, hm_ref,
               agla_ref, aglb_ref, agha_ref, aghb_ref, deg_ref,
               al_ref, ah_ref, am_ref, av_ref, out_ref):
    dinv = 1.0 / (1.0 + deg_ref[:, 0:1])
    hl = jnp.concatenate([hla_ref[...], hlb_ref[...]], axis=1)
    hh = jnp.concatenate([hha_ref[...], hhb_ref[...]], axis=1)
    aggl = jnp.concatenate([agla_ref[...], aglb_ref[...]], axis=1)
    aggh = jnp.concatenate([agha_ref[...], aghb_ref[...]], axis=1)
    ol = jnp.maximum(dinv * (hl + aggl), 0.0)
    oh = jnp.maximum(hh - dinv * (hh + aggh), 0.0)
    om = jnp.maximum(hm_ref[...], 0.0)
    s0 = jax.nn.sigmoid(jnp.dot(ol, al_ref[...],
                                preferred_element_type=jnp.float32))
    s1 = jax.nn.sigmoid(jnp.dot(oh, ah_ref[...],
                                preferred_element_type=jnp.float32))
    s2 = jax.nn.sigmoid(jnp.dot(om, am_ref[...],
                                preferred_element_type=jnp.float32))
    third = 1.0 / 3.0
    l0 = (s0 * av_ref[0, 0] + s1 * av_ref[1, 0] + s2 * av_ref[2, 0]) * third
    l1 = (s0 * av_ref[0, 1] + s1 * av_ref[1, 1] + s2 * av_ref[2, 1]) * third
    l2 = (s0 * av_ref[0, 2] + s1 * av_ref[1, 2] + s2 * av_ref[2, 2]) * third
    m = jnp.maximum(jnp.maximum(l0, l1), l2)
    e0 = jnp.exp(l0 - m)
    e1 = jnp.exp(l1 - m)
    e2 = jnp.exp(l2 - m)
    scale = 3.0 / (e0 + e1 + e2)
    out_ref[...] = scale * (e0 * ol + e1 * oh + e2 * om)


def _post(hla, hlb, hha, hhb, hm, agla, aglb, agha, aghb, deg,
          al, ah, am, av):
    grid = (N // BM,)
    row_spec = pl.BlockSpec((BM, D), lambda i: (i, 0))
    half_spec = pl.BlockSpec((BM, DH), lambda i: (i, 0))
    deg_spec = pl.BlockSpec((BM, 16), lambda i: (i, 0))
    a_spec = pl.BlockSpec((D, 1), lambda i: (0, 0))
    av_spec = pl.BlockSpec(memory_space=pltpu.SMEM)
    return pl.pallas_call(
        _post_body,
        grid=grid,
        in_specs=[half_spec, half_spec, half_spec, half_spec, row_spec,
                  half_spec, half_spec, half_spec, half_spec,
                  deg_spec, a_spec, a_spec, a_spec, av_spec],
        out_specs=row_spec,
        out_shape=jax.ShapeDtypeStruct((N, D), jnp.float32),
    )(hla, hlb, hha, hhb, hm, agla, aglb, agha, aghb, deg, al, ah, am, av)


def kernel(x, w_low0, w_high0, w_mlp0, a_low0, a_high0, a_mlp0, av0,
           w_low1, w_high1, w_mlp1, a_low1, a_high1, a_mlp1, av1,
           edge_index):
    row = edge_index[0].astype(jnp.int32)
    col = edge_index[1].astype(jnp.int32)
    row3 = jnp.pad(row, (0, E_PAD - E),
                   constant_values=N).reshape(NSUB, CHUNKS, CHUNK)
    col3 = jnp.pad(col, (0, E_PAD - E),
                   constant_values=0).reshape(NSUB, CHUNKS, CHUNK)
    zeros = jnp.zeros((N_PAD, DH), jnp.float32)
    zeros16 = jnp.zeros((N_PAD, 16), jnp.float32)
    ones16 = jnp.ones((CHUNK, 16), jnp.float32)

    hla0, hlb0, hha0, hhb0, hm0 = _pre(x, w_low0, w_high0, w_mlp0)
    agla0, aglb0, agha0, aghb0, deg = _sc_agg_deg(
        hla0, hlb0, hha0, hhb0, col3, row3, zeros, zeros16, ones16)
    degn = deg[:N]
    h1 = _post(hla0, hlb0, hha0, hhb0, hm0,
               agla0[:N], aglb0[:N], agha0[:N], aghb0[:N], degn,
               a_low0, a_high0, a_mlp0, av0)

    hla1, hlb1, hha1, hhb1, hm1 = _pre(h1, w_low1, w_high1, w_mlp1)
    agla1, aglb1, agha1, aghb1 = _sc_agg(
        hla1, hlb1, hha1, hhb1, col3, row3, zeros)
    out = _post(hla1, hlb1, hha1, hhb1, hm1,
                agla1[:N], aglb1[:N], agha1[:N], aghb1[:N], degn,
                a_low1, a_high1, a_mlp1, av1)
    return out


# speedup vs baseline: 1.0510x; 1.0510x over previous
import functools
import jax
import jax.numpy as jnp
from jax import lax
from jax.experimental import pallas as pl
from jax.experimental.pallas import tpu as pltpu
from jax.experimental.pallas import tpu_sc as plsc

N = 10000
D = 128
DH = 64
E = 320000
NSUB = 16
CHUNK = 128
CHUNKS = 158
TRIPLES = (CHUNKS - 5) // 3
E_SUB = CHUNKS * CHUNK
E_PAD = E_SUB * NSUB
N_PAD = 10112
RPW = N_PAD // NSUB
BM = 2000


def _make_sc_agg(with_deg):
    mesh = plsc.VectorSubcoreMesh(core_axis_name="c", subcore_axis_name="s")
    out_type = [jax.ShapeDtypeStruct((N_PAD, DH), jnp.float32)
                for _ in range(4)]
    scratch = [
        pltpu.VMEM((CHUNKS, CHUNK), jnp.int32),
        pltpu.VMEM((CHUNKS, CHUNK), jnp.int32),
        pltpu.VMEM((CHUNK, DH), jnp.float32),
        pltpu.VMEM((CHUNK, DH), jnp.float32),
        pltpu.VMEM((CHUNK, DH), jnp.float32),
        pltpu.VMEM_SHARED((N_PAD, DH), jnp.float32),
    ] + [pltpu.SemaphoreType.DMA] * 6
    if with_deg:
        out_type.append(jax.ShapeDtypeStruct((N_PAD, 16), jnp.float32))
        scratch += [
            pltpu.VMEM((CHUNK, 16), jnp.float32),
            pltpu.VMEM_SHARED((N_PAD, 16), jnp.float32),
        ]

    @functools.partial(
        pl.kernel, mesh=mesh, out_type=out_type, scratch_types=scratch,
        compiler_params=pltpu.CompilerParams(use_tc_tiling_on_sc=False))
    def sc_agg(*refs):
        if with_deg:
            (hl_a, hl_b, hh_a, hh_b, col3, row3, zeros_hbm, zeros16_hbm,
             ones_hbm, aggl_a, aggl_b, aggh_a, aggh_b, deg_out,
             col_v, row_v, rows_a, rows_b, rows_c, acc,
             g0, g1, g2, s0, s1, s2, ones_v, accd) = refs
        else:
            (hl_a, hl_b, hh_a, hh_b, col3, row3, zeros_hbm,
             aggl_a, aggl_b, aggh_a, aggh_b,
             col_v, row_v, rows_a, rows_b, rows_c, acc,
             g0, g1, g2, s0, s1, s2) = refs
        bufs = (rows_a, rows_b, rows_c)
        gsems = (g0, g1, g2)
        ssems = (s0, s1, s2)

        cid = lax.axis_index("c")
        sid = lax.axis_index("s")
        sl = pl.ds(sid * RPW, RPW)

        pltpu.sync_copy(col3.at[sid], col_v)
        pltpu.sync_copy(row3.at[sid], row_v)
        pltpu.sync_copy(zeros_hbm.at[sl], acc.at[sl])
        if with_deg:
            pltpu.sync_copy(zeros16_hbm.at[sl], accd.at[sl])
            pltpu.sync_copy(ones_hbm, ones_v)
        plsc.subcore_barrier()

        def run(src):
            dummy = src.at[pl.ds(0, CHUNK)]

            def gfire(j, x):
                pltpu.async_copy(src.at[col_v.at[j]], bufs[x], gsems[x])

            def gwait(x):
                pltpu.make_async_copy(dummy, bufs[x], gsems[x]).wait()

            def sfire(j, x):
                pltpu.async_copy(bufs[x], acc.at[row_v.at[j]], ssems[x],
                                 add=True)

            def swait(x):
                pltpu.make_async_copy(bufs[x], acc.at[pl.ds(0, CHUNK)],
                                      ssems[x]).wait()

            gfire(0, 0)
            gfire(1, 1)
            gwait(0); sfire(0, 0); gfire(2, 2)
            gwait(1); sfire(1, 1); swait(0); gfire(3, 0)

            def step(t, carry):
                j = 3 * t + 2
                for d in range(3):
                    x = (2 + d) % 3
                    gwait(x)
                    sfire(j + d, x)
                    swait((x + 2) % 3)
                    gfire(j + d + 2, (x + 2) % 3)
                return carry

            lax.fori_loop(0, TRIPLES, step, 0)
            gwait(2); sfire(CHUNKS - 3, 2); swait(1); gfire(CHUNKS - 1, 1)
            gwait(0); sfire(CHUNKS - 2, 0)
            gwait(1); sfire(CHUNKS - 1, 1)
            swait(2); swait(0); swait(1)

        for p, (srcs, outs) in enumerate((((hl_a, hh_a), (aggl_a, aggh_a)),
                                          ((hl_b, hh_b), (aggl_b, aggh_b)))):
            @pl.when(cid == 0)
            def _(srcs=srcs):
                run(srcs[0])

            @pl.when(cid == 1)
            def _(srcs=srcs):
                run(srcs[1])

            if with_deg and p == 0:
                @pl.when(cid == 0)
                def _():
                    def dstep(j, carry):
                        pltpu.sync_copy(ones_v, accd.at[row_v.at[j]],
                                        add=True)
                        return carry
                    lax.fori_loop(0, CHUNKS, dstep, 0)

            plsc.subcore_barrier()

            @pl.when(cid == 0)
            def _(outs=outs):
                pltpu.sync_copy(acc.at[sl], outs[0].at[sl])

            @pl.when(cid == 1)
            def _(outs=outs):
                pltpu.sync_copy(acc.at[sl], outs[1].at[sl])

            if with_deg and p == 0:
                @pl.when(cid == 0)
                def _():
                    pltpu.sync_copy(accd.at[sl], deg_out.at[sl])

            if p == 0:
                pltpu.sync_copy(zeros_hbm.at[sl], acc.at[sl])
                plsc.subcore_barrier()

    return sc_agg


_sc_agg_deg = _make_sc_agg(True)
_sc_agg = _make_sc_agg(False)


def _pre_body(x_ref, wl_ref, wh_ref, wm_ref,
              la_ref, lb_ref, ha_ref, hb_ref, om_ref):
    h = x_ref[...]
    rl = jnp.dot(h, wl_ref[...], preferred_element_type=jnp.float32)
    rh = jnp.dot(h, wh_ref[...], preferred_element_type=jnp.float32)
    la_ref[...] = rl[:, :DH]
    lb_ref[...] = rl[:, DH:]
    ha_ref[...] = rh[:, :DH]
    hb_ref[...] = rh[:, DH:]
    om_ref[...] = jnp.dot(h, wm_ref[...], preferred_element_type=jnp.float32)


def _pre(x, wl, wh, wm):
    grid = (N // BM,)
    row_spec = pl.BlockSpec((BM, D), lambda i: (i, 0))
    half_spec = pl.BlockSpec((BM, DH), lambda i: (i, 0))
    w_spec = pl.BlockSpec((D, D), lambda i: (0, 0))
    return pl.pallas_call(
        _pre_body,
        grid=grid,
        in_specs=[row_spec, w_spec, w_spec, w_spec],
        out_specs=[half_spec, half_spec, half_spec, half_spec, row_spec],
        out_shape=[jax.ShapeDtypeStruct((N, DH), jnp.float32)] * 4
        + [jax.ShapeDtypeStruct((N, D), jnp.float32)],
    )(x, wl, wh, wm)


def _post_body(hla_ref, hlb_ref, hha_ref, hhb_ref, hm_ref,
               agla_ref, aglb_ref, agha_ref, aghb_ref, deg_ref,
               al_ref, ah_ref, am_ref, av_ref, out_ref):
    dinv = 1.0 / (1.0 + deg_ref[:, 0:1])
    hl = jnp.concatenate([hla_ref[...], hlb_ref[...]], axis=1)
    hh = jnp.concatenate([hha_ref[...], hhb_ref[...]], axis=1)
    aggl = jnp.concatenate([agla_ref[...], aglb_ref[...]], axis=1)
    aggh = jnp.concatenate([agha_ref[...], aghb_ref[...]], axis=1)
    ol = jnp.maximum(dinv * (hl + aggl), 0.0)
    oh = jnp.maximum(hh - dinv * (hh + aggh), 0.0)
    om = jnp.maximum(hm_ref[...], 0.0)
    s0 = jax.nn.sigmoid(jnp.dot(ol, al_ref[...],
                                preferred_element_type=jnp.float32))
    s1 = jax.nn.sigmoid(jnp.dot(oh, ah_ref[...],
                                preferred_element_type=jnp.float32))
    s2 = jax.nn.sigmoid(jnp.dot(om, am_ref[...],
                                preferred_element_type=jnp.float32))
    third = 1.0 / 3.0
    l0 = (s0 * av_ref[0, 0] + s1 * av_ref[1, 0] + s2 * av_ref[2, 0]) * third
    l1 = (s0 * av_ref[0, 1] + s1 * av_ref[1, 1] + s2 * av_ref[2, 1]) * third
    l2 = (s0 * av_ref[0, 2] + s1 * av_ref[1, 2] + s2 * av_ref[2, 2]) * third
    m = jnp.maximum(jnp.maximum(l0, l1), l2)
    e0 = jnp.exp(l0 - m)
    e1 = jnp.exp(l1 - m)
    e2 = jnp.exp(l2 - m)
    scale = 3.0 / (e0 + e1 + e2)
    out_ref[...] = scale * (e0 * ol + e1 * oh + e2 * om)


def _post(hla, hlb, hha, hhb, hm, agla, aglb, agha, aghb, deg,
          al, ah, am, av):
    grid = (N // BM,)
    row_spec = pl.BlockSpec((BM, D), lambda i: (i, 0))
    half_spec = pl.BlockSpec((BM, DH), lambda i: (i, 0))
    deg_spec = pl.BlockSpec((BM, 16), lambda i: (i, 0))
    a_spec = pl.BlockSpec((D, 1), lambda i: (0, 0))
    av_spec = pl.BlockSpec(memory_space=pltpu.SMEM)
    return pl.pallas_call(
        _post_body,
        grid=grid,
        in_specs=[half_spec, half_spec, half_spec, half_spec, row_spec,
                  half_spec, half_spec, half_spec, half_spec,
                  deg_spec, a_spec, a_spec, a_spec, av_spec],
        out_specs=row_spec,
        out_shape=jax.ShapeDtypeStruct((N, D), jnp.float32),
    )(hla, hlb, hha, hhb, hm, agla, aglb, agha, aghb, deg, al, ah, am, av)


def kernel(x, w_low0, w_high0, w_mlp0, a_low0, a_high0, a_mlp0, av0,
           w_low1, w_high1, w_mlp1, a_low1, a_high1, a_mlp1, av1,
           edge_index):
    row = edge_index[0].astype(jnp.int32)
    col = edge_index[1].astype(jnp.int32)
    row3 = jnp.pad(row, (0, E_PAD - E),
                   constant_values=N).reshape(NSUB, CHUNKS, CHUNK)
    col3 = jnp.pad(col, (0, E_PAD - E),
                   constant_values=0).reshape(NSUB, CHUNKS, CHUNK)
    zeros = jnp.zeros((N_PAD, DH), jnp.float32)
    zeros16 = jnp.zeros((N_PAD, 16), jnp.float32)
    ones16 = jnp.ones((CHUNK, 16), jnp.float32)

    hla0, hlb0, hha0, hhb0, hm0 = _pre(x, w_low0, w_high0, w_mlp0)
    agla0, aglb0, agha0, aghb0, deg = _sc_agg_deg(
        hla0, hlb0, hha0, hhb0, col3, row3, zeros, zeros16, ones16)
    degn = deg[:N]
    h1 = _post(hla0, hlb0, hha0, hhb0, hm0,
               agla0[:N], aglb0[:N], agha0[:N], aghb0[:N], degn,
               a_low0, a_high0, a_mlp0, av0)

    hla1, hlb1, hha1, hhb1, hm1 = _pre(h1, w_low1, w_high1, w_mlp1)
    agla1, aglb1, agha1, aghb1 = _sc_agg(
        hla1, hlb1, hha1, hhb1, col3, row3, zeros)
    out = _post(hla1, hlb1, hha1, hhb1, hm1,
                agla1[:N], aglb1[:N], agha1[:N], aghb1[:N], degn,
                a_low1, a_high1, a_mlp1, av1)
    return out
